# Initial kernel scaffold; baseline (speedup 1.0000x reference)
import functools

import jax
import jax.numpy as jnp
from jax import lax
from jax.experimental import pallas as pl
from jax.experimental.pallas import tpu as pltpu
from jax.experimental.pallas import tpu_sc as plsc

F32 = jnp.float32
_EMB = 64
_LANES = 16
_LA, _LB = 104, 96


def _pool_sc(shapes, colors, clusters, shape_emb, color_emb, cluster_emb):
    B, L = shapes.shape
    assert L == _LA + _LB
    info = plsc.get_sparse_core_info()
    NC, NS = info.num_cores, info.num_subcores
    NW = NC * NS
    RPW = B // NW

    mesh = plsc.VectorSubcoreMesh(core_axis_name="c", subcore_axis_name="s")

    @functools.partial(
        pl.kernel,
        out_type=jax.ShapeDtypeStruct((B, 3 * _EMB), F32),
        mesh=mesh,
        scratch_types=[
            pltpu.VMEM((3, _LA), jnp.int32),
            pltpu.VMEM((3, _LB), jnp.int32),
            pltpu.VMEM((3, L, _EMB), F32),
            pltpu.VMEM((RPW, 3 * _EMB), F32),
            pltpu.VMEM((3, _EMB), F32),
            pltpu.SemaphoreType.DMA,
        ],
    )
    def k(shapes_h, colors_h, clusters_h, se_h, ce_h, ue_h, out_h,
          idxA, idxB, buf, pooled, t0, sem):
        wid = lax.axis_index("s") * NC + lax.axis_index("c")
        base = wid * RPW
        idx_hs = (shapes_h, colors_h, clusters_h)
        tab_hs = (se_h, ce_h, ue_h)

        for t in range(3):
            pltpu.sync_copy(tab_hs[t].at[0], t0.at[t])

        lane = lax.iota(jnp.int32, _LANES)

        def row_body(r, carry):
            row = base + r
            for t in range(3):
                pltpu.sync_copy(idx_hs[t].at[row, pl.ds(0, _LA)], idxA.at[t])
                pltpu.sync_copy(idx_hs[t].at[row, pl.ds(_LA, _LB)], idxB.at[t])
            cps = []
            for t in range(3):
                cps.append(pltpu.async_copy(
                    tab_hs[t].at[idxA.at[t]], buf.at[t, pl.ds(0, _LA)], sem))
                cps.append(pltpu.async_copy(
                    tab_hs[t].at[idxB.at[t]], buf.at[t, pl.ds(_LA, _LB)], sem))
            for cp in cps:
                cp.wait()

            def tok_body(l, accs):
                out = []
                for t in range(3):
                    for c in range(4):
                        v = buf[t, l, pl.ds(c * _LANES, _LANES)]
                        out.append(accs[t * 4 + c] + v)
                return tuple(out)

            accs = lax.fori_loop(
                0, L, tok_body,
                tuple(jnp.zeros((_LANES,), F32) for _ in range(12)))

            for t in range(3):
                cnt = jnp.zeros((_LANES,), F32)
                for kk in range(6):
                    iv = idxA[t, pl.ds(kk * _LANES, _LANES)]
                    cnt = cnt + (iv == 0).astype(F32)
                iv = idxA[t, pl.ds(_LA - _LANES, _LANES)]
                cnt = cnt + ((iv == 0) & (lane >= 8)).astype(F32)
                for kk in range(6):
                    iv = idxB[t, pl.ds(kk * _LANES, _LANES)]
                    cnt = cnt + (iv == 0).astype(F32)
                n0 = jnp.sum(cnt)
                for c in range(4):
                    pooled[r, pl.ds((t * 4 + c) * _LANES, _LANES)] = (
                        accs[t * 4 + c] - n0 * t0[t, pl.ds(c * _LANES, _LANES)])
            return carry

        lax.fori_loop(0, RPW, row_body, 0)
        pltpu.sync_copy(pooled, out_h.at[pl.ds(base, RPW)])

    return k(shapes, colors, clusters, shape_emb, color_emb, cluster_emb)


def _mlp_tc(pooled, mask, W1, b1, W2, b2):
    B = pooled.shape[0]
    NCLS = W2.shape[1]

    def k(pooled_ref, mask_ref, w1_ref, b1_ref, w2_ref, b2_ref, out_ref):
        ms = jnp.sum(mask_ref[...], axis=1, keepdims=True)
        p = pooled_ref[...] / ms
        h = jnp.dot(p, w1_ref[...], preferred_element_type=F32) + b1_ref[...]
        h = jnp.maximum(h, 0.0)
        out_ref[...] = jnp.dot(h, w2_ref[...], preferred_element_type=F32) + b2_ref[...]

    return pl.pallas_call(
        k,
        out_shape=jax.ShapeDtypeStruct((B, NCLS), F32),
    )(pooled, mask, W1, b1.reshape(1, -1), W2, b2.reshape(1, -1))


def kernel(shapes, colors, clusters, mask, shape_emb, color_emb, cluster_emb,
           W1, b1, W2, b2):
    pooled = _pool_sc(shapes, colors, clusters,
                      shape_emb, color_emb, cluster_emb)
    return _mlp_tc(pooled, mask, W1, b1, W2, b2)


# speedup vs baseline: 10.7389x; 10.7389x over previous
import functools

import jax
import jax.numpy as jnp
from jax import lax
from jax.experimental import pallas as pl
from jax.experimental.pallas import tpu as pltpu
from jax.experimental.pallas import tpu_sc as plsc

F32 = jnp.float32
_EMB = 64
_LANES = 16
_LA, _LB = 104, 96


def _pool_sc(shapes, colors, clusters, shape_emb, color_emb, cluster_emb):
    B, L = shapes.shape
    assert L == _LA + _LB
    info = plsc.get_sparse_core_info()
    NC, NS = info.num_cores, info.num_subcores
    NW = NC * NS
    RPW = B // NW

    mesh = plsc.VectorSubcoreMesh(core_axis_name="c", subcore_axis_name="s")

    @functools.partial(
        pl.kernel,
        out_type=jax.ShapeDtypeStruct((B, 3 * _EMB), F32),
        mesh=mesh,
        compiler_params=pltpu.CompilerParams(use_tc_tiling_on_sc=False,
                                              needs_layout_passes=False),
        scratch_types=[
            pltpu.VMEM((3, _LA), jnp.int32),
            pltpu.VMEM((3, _LB), jnp.int32),
            pltpu.VMEM((3, L, _EMB), F32),
            pltpu.VMEM((RPW, 3 * _EMB), F32),
            pltpu.VMEM((3, _EMB), F32),
            pltpu.SemaphoreType.DMA,
        ],
    )
    def k(shapes_h, colors_h, clusters_h, se_h, ce_h, ue_h, out_h,
          idxA, idxB, buf, pooled, t0, sem):
        wid = lax.axis_index("s") * NC + lax.axis_index("c")
        base = wid * RPW
        idx_hs = (shapes_h, colors_h, clusters_h)
        tab_hs = (se_h, ce_h, ue_h)

        for t in range(3):
            pltpu.sync_copy(tab_hs[t].at[0], t0.at[t])

        lane = lax.iota(jnp.int32, _LANES)
        zeros = jnp.zeros((_LANES,), F32)
        ones = zeros + 1.0
        tailf = jnp.where(lane >= 8, ones, zeros)

        def row_body(r, carry):
            row = base + r
            off = pl.multiple_of(row * L, 8)
            offb = pl.multiple_of(row * L + _LA, 8)
            for t in range(3):
                pltpu.sync_copy(idx_hs[t].at[pl.ds(off, _LA)], idxA.at[t])
                pltpu.sync_copy(idx_hs[t].at[pl.ds(offb, _LB)], idxB.at[t])
            cps = []
            for t in range(3):
                cps.append(pltpu.async_copy(
                    tab_hs[t].at[idxA.at[t]], buf.at[t, pl.ds(0, _LA)], sem))
                cps.append(pltpu.async_copy(
                    tab_hs[t].at[idxB.at[t]], buf.at[t, pl.ds(_LA, _LB)], sem))
            for cp in cps:
                cp.wait()

            def tok_body(l, accs):
                out = []
                for t in range(3):
                    for c in range(4):
                        v = buf[t, l, pl.ds(c * _LANES, _LANES)]
                        out.append(accs[t * 4 + c] + v)
                return tuple(out)

            accs = lax.fori_loop(
                0, L, tok_body,
                tuple(jnp.zeros((_LANES,), F32) for _ in range(12)))

            for t in range(3):
                cnt = zeros
                for kk in range(6):
                    iv = idxA[t, pl.ds(kk * _LANES, _LANES)]
                    cnt = cnt + jnp.where(iv == 0, ones, zeros)
                iv = idxA[t, pl.ds(_LA - _LANES, _LANES)]
                cnt = cnt + jnp.where(iv == 0, tailf, zeros)
                for kk in range(6):
                    iv = idxB[t, pl.ds(kk * _LANES, _LANES)]
                    cnt = cnt + jnp.where(iv == 0, ones, zeros)
                n0 = lax.broadcast_in_dim(jnp.sum(cnt), (_LANES,), ())
                for c in range(4):
                    pooled[r, pl.ds((t * 4 + c) * _LANES, _LANES)] = (
                        accs[t * 4 + c] - n0 * t0[t, pl.ds(c * _LANES, _LANES)])
            return carry

        lax.fori_loop(0, RPW, row_body, 0)
        pltpu.sync_copy(pooled, out_h.at[pl.ds(pl.multiple_of(base, 8), RPW)])

    return k(shapes.reshape(-1), colors.reshape(-1), clusters.reshape(-1),
             shape_emb, color_emb, cluster_emb)


def _mlp_tc(pooled, mask, W1, b1, W2, b2):
    B = pooled.shape[0]
    NCLS = W2.shape[1]

    def k(pooled_ref, mask_ref, w1_ref, b1_ref, w2_ref, b2_ref, out_ref):
        ms = jnp.sum(mask_ref[...], axis=1, keepdims=True)
        p = pooled_ref[...] / ms
        h = jnp.dot(p, w1_ref[...], preferred_element_type=F32) + b1_ref[...]
        h = jnp.maximum(h, 0.0)
        out_ref[...] = jnp.dot(h, w2_ref[...], preferred_element_type=F32) + b2_ref[...]

    return pl.pallas_call(
        k,
        out_shape=jax.ShapeDtypeStruct((B, NCLS), F32),
    )(pooled, mask, W1, b1.reshape(1, -1), W2, b2.reshape(1, -1))


def kernel(shapes, colors, clusters, mask, shape_emb, color_emb, cluster_emb,
           W1, b1, W2, b2):
    pooled = _pool_sc(shapes, colors, clusters,
                      shape_emb, color_emb, cluster_emb)
    return _mlp_tc(pooled, mask, W1, b1, W2, b2)


# speedup vs baseline: 21.9157x; 2.0408x over previous
import functools

import jax
import jax.numpy as jnp
from jax import lax
from jax.experimental import pallas as pl
from jax.experimental.pallas import tpu as pltpu
from jax.experimental.pallas import tpu_sc as plsc

F32 = jnp.float32
_EMB = 64
_LANES = 16
_LA, _LB = 104, 96
_CH = 16


def _pool_sc(shapes, colors, clusters, shape_emb, color_emb, cluster_emb):
    B, L = shapes.shape
    assert L == _LA + _LB
    info = plsc.get_sparse_core_info()
    NC, NS = info.num_cores, info.num_subcores
    NW = NC * NS
    RPW = B // NW
    NCHUNK = RPW // _CH

    mesh = plsc.VectorSubcoreMesh(core_axis_name="c", subcore_axis_name="s")

    @functools.partial(
        pl.kernel,
        out_type=jax.ShapeDtypeStruct((B, 3 * _EMB), F32),
        mesh=mesh,
        compiler_params=pltpu.CompilerParams(use_tc_tiling_on_sc=False,
                                             needs_layout_passes=False),
        scratch_types=[
            pltpu.VMEM((2, 3, _CH, _LA), jnp.int32),
            pltpu.VMEM((2, 3, _CH, _LB), jnp.int32),
            pltpu.VMEM((2, 3, L, _EMB), F32),
            pltpu.VMEM((RPW, 3 * _EMB), F32),
            pltpu.VMEM((3, _EMB), F32),
            pltpu.SemaphoreType.DMA((2,)),
            pltpu.SemaphoreType.DMA((2,)),
        ],
    )
    def k(shapes_h, colors_h, clusters_h, se_h, ce_h, ue_h, out_h,
          idxA, idxB, buf, pooled, t0, isem, gsem):
        wid = lax.axis_index("s") * NC + lax.axis_index("c")
        base = wid * RPW
        idx_hs = (shapes_h, colors_h, clusters_h)
        tab_hs = (se_h, ce_h, ue_h)

        for t in range(3):
            pltpu.sync_copy(tab_hs[t].at[0], t0.at[t])

        lane = lax.iota(jnp.int32, _LANES)
        zeros = jnp.zeros((_LANES,), F32)
        ones = zeros + 1.0
        tailf = jnp.where(lane >= 8, ones, zeros)

        def stage_chunk(c, slot):
            row0 = pl.multiple_of(base + c * _CH, 8)
            for t in range(3):
                pltpu.async_copy(
                    idx_hs[t].at[pl.ds(row0, _CH), pl.ds(0, _LA)],
                    idxA.at[slot, t], isem.at[slot])
                pltpu.async_copy(
                    idx_hs[t].at[pl.ds(row0, _CH), pl.ds(_LA, _LB)],
                    idxB.at[slot, t], isem.at[slot])

        def wait_chunk(slot):
            for t in range(3):
                pltpu.make_async_copy(
                    idx_hs[t].at[pl.ds(0, _CH), pl.ds(0, _LA)],
                    idxA.at[slot, t], isem.at[slot]).wait()
                pltpu.make_async_copy(
                    idx_hs[t].at[pl.ds(0, _CH), pl.ds(_LA, _LB)],
                    idxB.at[slot, t], isem.at[slot]).wait()

        def launch_gathers(r, gslot, cslot):
            rr = r - (r // _CH) * _CH
            for t in range(3):
                pltpu.async_copy(
                    tab_hs[t].at[idxA.at[cslot, t, rr]],
                    buf.at[gslot, t, pl.ds(0, _LA)], gsem.at[gslot])
                pltpu.async_copy(
                    tab_hs[t].at[idxB.at[cslot, t, rr]],
                    buf.at[gslot, t, pl.ds(_LA, _LB)], gsem.at[gslot])

        def wait_gathers(gslot, cslot):
            for t in range(3):
                pltpu.make_async_copy(
                    tab_hs[t].at[idxA.at[cslot, t, 0]],
                    buf.at[gslot, t, pl.ds(0, _LA)], gsem.at[gslot]).wait()
                pltpu.make_async_copy(
                    tab_hs[t].at[idxB.at[cslot, t, 0]],
                    buf.at[gslot, t, pl.ds(_LA, _LB)], gsem.at[gslot]).wait()

        stage_chunk(0, 0)
        stage_chunk(1, 1)
        wait_chunk(0)
        launch_gathers(0, 0, 0)

        def row_body(r, carry):
            gslot = lax.rem(r, 2)
            chunk = r // _CH
            cslot = lax.rem(chunk, 2)
            rr = r - chunk * _CH

            nxt = r + 1

            @pl.when(nxt < RPW)
            def _():
                nchunk = nxt // _CH
                ncslot = lax.rem(nchunk, 2)

                @pl.when(nxt == nchunk * _CH)
                def _():
                    wait_chunk(ncslot)

                launch_gathers(nxt, lax.rem(nxt, 2), ncslot)

            wait_gathers(gslot, cslot)

            def tok_body(l, accs):
                out = []
                for t in range(3):
                    for c in range(4):
                        v = buf[gslot, t, l, pl.ds(c * _LANES, _LANES)]
                        out.append(accs[t * 4 + c] + v)
                return tuple(out)

            accs = lax.fori_loop(
                0, L, tok_body,
                tuple(jnp.zeros((_LANES,), F32) for _ in range(12)))

            for t in range(3):
                cnt = zeros
                for kk in range(6):
                    iv = idxA[cslot, t, rr, pl.ds(kk * _LANES, _LANES)]
                    cnt = cnt + jnp.where(iv == 0, ones, zeros)
                iv = idxA[cslot, t, rr, pl.ds(_LA - _LANES, _LANES)]
                cnt = cnt + jnp.where(iv == 0, tailf, zeros)
                for kk in range(6):
                    iv = idxB[cslot, t, rr, pl.ds(kk * _LANES, _LANES)]
                    cnt = cnt + jnp.where(iv == 0, ones, zeros)
                n0 = lax.broadcast_in_dim(jnp.sum(cnt), (_LANES,), ())
                for c in range(4):
                    pooled[r, pl.ds((t * 4 + c) * _LANES, _LANES)] = (
                        accs[t * 4 + c] - n0 * t0[t, pl.ds(c * _LANES, _LANES)])

            @pl.when((rr == _CH - 1) & (chunk + 2 < NCHUNK))
            def _():
                stage_chunk(chunk + 2, cslot)

            return carry

        lax.fori_loop(0, RPW, row_body, 0)
        pltpu.sync_copy(pooled, out_h.at[pl.ds(pl.multiple_of(base, 8), RPW)])

    return k(shapes, colors, clusters, shape_emb, color_emb, cluster_emb)


def _mlp_tc(pooled, mask, W1, b1, W2, b2):
    B = pooled.shape[0]
    NCLS = W2.shape[1]

    def k(pooled_ref, mask_ref, w1_ref, b1_ref, w2_ref, b2_ref, out_ref):
        ms = jnp.sum(mask_ref[...], axis=1, keepdims=True)
        p = pooled_ref[...] / ms
        h = jnp.dot(p, w1_ref[...], preferred_element_type=F32) + b1_ref[...]
        h = jnp.maximum(h, 0.0)
        out_ref[...] = jnp.dot(h, w2_ref[...], preferred_element_type=F32) + b2_ref[...]

    return pl.pallas_call(
        k,
        out_shape=jax.ShapeDtypeStruct((B, NCLS), F32),
    )(pooled, mask, W1, b1.reshape(1, -1), W2, b2.reshape(1, -1))


def kernel(shapes, colors, clusters, mask, shape_emb, color_emb, cluster_emb,
           W1, b1, W2, b2):
    pooled = _pool_sc(shapes, colors, clusters,
                      shape_emb, color_emb, cluster_emb)
    return _mlp_tc(pooled, mask, W1, b1, W2, b2)


# speedup vs baseline: 31.1818x; 1.4228x over previous
import functools

import jax
import jax.numpy as jnp
from jax import lax
from jax.experimental import pallas as pl
from jax.experimental.pallas import tpu as pltpu
from jax.experimental.pallas import tpu_sc as plsc

F32 = jnp.float32
_EMB = 64
_LANES = 16
_LA, _LB = 104, 96
_CH = 16
_NBIN = 1008


def _pool_sc(shapes, colors, clusters, cluster_emb):
    B, L = shapes.shape
    assert L == _LA + _LB
    info = plsc.get_sparse_core_info()
    NC, NS = info.num_cores, info.num_subcores
    NW = NC * NS
    RPW = B // NW
    NCHUNK = RPW // _CH

    mesh = plsc.VectorSubcoreMesh(core_axis_name="c", subcore_axis_name="s")

    @functools.partial(
        pl.kernel,
        out_type=(jax.ShapeDtypeStruct((B, _EMB), F32),
                  jax.ShapeDtypeStruct((B, _NBIN), F32),
                  jax.ShapeDtypeStruct((B, _NBIN), F32)),
        mesh=mesh,
        compiler_params=pltpu.CompilerParams(use_tc_tiling_on_sc=False,
                                             needs_layout_passes=False),
        scratch_types=[
            pltpu.VMEM((2, 3, _CH, _LA), jnp.int32),
            pltpu.VMEM((2, 3, _CH, _LB), jnp.int32),
            pltpu.VMEM((2, L, _EMB), F32),
            pltpu.VMEM((RPW, _EMB), F32),
            pltpu.VMEM((_EMB,), F32),
            pltpu.VMEM((2, 2, _NBIN), F32),
            pltpu.SemaphoreType.DMA((2,)),
            pltpu.SemaphoreType.DMA((2,)),
            pltpu.SemaphoreType.DMA((2,)),
        ],
    )
    def k(shapes_h, colors_h, clusters_h, ue_h, pool_h, cs_h, cc_h,
          idxA, idxB, buf, pooled, t0, cnts, isem, gsem, csem):
        wid = lax.axis_index("s") * NC + lax.axis_index("c")
        base = wid * RPW
        idx_hs = (shapes_h, colors_h, clusters_h)
        cnt_hs = (cs_h, cc_h)

        pltpu.sync_copy(ue_h.at[0], t0)

        lane = lax.iota(jnp.int32, _LANES)
        zeros = jnp.zeros((_LANES,), F32)
        ones = zeros + 1.0
        tailm = lane >= 8
        tailf = jnp.where(tailm, ones, zeros)

        def stage_chunk(c, slot):
            row0 = pl.multiple_of(base + c * _CH, 8)
            for t in range(3):
                pltpu.async_copy(
                    idx_hs[t].at[pl.ds(row0, _CH), pl.ds(0, _LA)],
                    idxA.at[slot, t], isem.at[slot])
                pltpu.async_copy(
                    idx_hs[t].at[pl.ds(row0, _CH), pl.ds(_LA, _LB)],
                    idxB.at[slot, t], isem.at[slot])

        def wait_chunk(slot):
            for t in range(3):
                pltpu.make_async_copy(
                    idx_hs[t].at[pl.ds(0, _CH), pl.ds(0, _LA)],
                    idxA.at[slot, t], isem.at[slot]).wait()
                pltpu.make_async_copy(
                    idx_hs[t].at[pl.ds(0, _CH), pl.ds(_LA, _LB)],
                    idxB.at[slot, t], isem.at[slot]).wait()

        def launch_gathers(r, gslot, cslot):
            rr = r - (r // _CH) * _CH
            pltpu.async_copy(ue_h.at[idxA.at[cslot, 2, rr]],
                             buf.at[gslot, pl.ds(0, _LA)], gsem.at[gslot])
            pltpu.async_copy(ue_h.at[idxB.at[cslot, 2, rr]],
                             buf.at[gslot, pl.ds(_LA, _LB)], gsem.at[gslot])

        def wait_gathers(gslot, cslot):
            pltpu.make_async_copy(ue_h.at[idxA.at[cslot, 2, 0]],
                                  buf.at[gslot, pl.ds(0, _LA)],
                                  gsem.at[gslot]).wait()
            pltpu.make_async_copy(ue_h.at[idxB.at[cslot, 2, 0]],
                                  buf.at[gslot, pl.ds(_LA, _LB)],
                                  gsem.at[gslot]).wait()

        stage_chunk(0, 0)
        stage_chunk(1, 1)
        wait_chunk(0)
        launch_gathers(0, 0, 0)

        def row_body(r, carry):
            gslot = lax.rem(r, 2)
            chunk = r // _CH
            cslot = lax.rem(chunk, 2)
            rr = r - chunk * _CH
            row = base + r

            nxt = r + 1

            @pl.when(nxt < RPW)
            def _():
                nchunk = nxt // _CH
                ncslot = lax.rem(nchunk, 2)

                @pl.when(nxt == nchunk * _CH)
                def _():
                    wait_chunk(ncslot)

                launch_gathers(nxt, lax.rem(nxt, 2), ncslot)

            wait_gathers(gslot, cslot)

            def tok_body(l, accs):
                return tuple(
                    accs[c] + buf[gslot, l, pl.ds(c * _LANES, _LANES)]
                    for c in range(4))

            accs = lax.fori_loop(
                0, L, tok_body, tuple(zeros for _ in range(4)))

            cnt = zeros
            for kk in range(6):
                iv = idxA[cslot, 2, rr, pl.ds(kk * _LANES, _LANES)]
                cnt = cnt + jnp.where(iv == 0, ones, zeros)
            iv = idxA[cslot, 2, rr, pl.ds(_LA - _LANES, _LANES)]
            cnt = cnt + jnp.where(iv == 0, tailf, zeros)
            for kk in range(6):
                iv = idxB[cslot, 2, rr, pl.ds(kk * _LANES, _LANES)]
                cnt = cnt + jnp.where(iv == 0, ones, zeros)
            n0 = lax.broadcast_in_dim(jnp.sum(cnt), (_LANES,), ())
            for c in range(4):
                pooled[r, pl.ds(c * _LANES, _LANES)] = (
                    accs[c] - n0 * t0[pl.ds(c * _LANES, _LANES)])

            @pl.when(r >= 2)
            def _():
                for t in range(2):
                    pltpu.make_async_copy(
                        cnts.at[gslot, t], cnt_hs[t].at[0],
                        csem.at[gslot]).wait()

            for t in range(2):
                cref = cnts.at[gslot, t]
                for kk in range(_NBIN // _LANES):
                    cref[pl.ds(kk * _LANES, _LANES)] = zeros
                for kk in range(6):
                    iv = idxA[cslot, t, rr, pl.ds(kk * _LANES, _LANES)]
                    plsc.addupdate_scatter(cref, [iv], ones)
                iv = idxA[cslot, t, rr, pl.ds(_LA - _LANES, _LANES)]
                plsc.addupdate_scatter(cref, [iv], ones, mask=tailm)
                for kk in range(6):
                    iv = idxB[cslot, t, rr, pl.ds(kk * _LANES, _LANES)]
                    plsc.addupdate_scatter(cref, [iv], ones)
                pltpu.async_copy(cnts.at[gslot, t], cnt_hs[t].at[row],
                                 csem.at[gslot])

            @pl.when((rr == _CH - 1) & (chunk + 2 < NCHUNK))
            def _():
                stage_chunk(chunk + 2, cslot)

            return carry

        lax.fori_loop(0, RPW, row_body, 0)

        for s in range(2):
            for t in range(2):
                pltpu.make_async_copy(
                    cnts.at[s, t], cnt_hs[t].at[0], csem.at[s]).wait()

        pltpu.sync_copy(pooled, pool_h.at[pl.ds(pl.multiple_of(base, 8), RPW)])

    return k(shapes, colors, clusters, cluster_emb)


def _mlp_tc(pool_u, cs, cc, mask, se_pad, ce_pad, W1, b1, W2, b2):
    B = pool_u.shape[0]
    NCLS = W2.shape[1]

    def k(pu_ref, cs_ref, cc_ref, mask_ref, se_ref, ce_ref,
          w1_ref, b1_ref, w2_ref, b2_ref, out_ref):
        ps = jnp.dot(cs_ref[...], se_ref[...], preferred_element_type=F32)
        pc = jnp.dot(cc_ref[...], ce_ref[...], preferred_element_type=F32)
        pooled = jnp.concatenate([ps, pc, pu_ref[...]], axis=1)
        ms = jnp.sum(mask_ref[...], axis=1, keepdims=True)
        h = jnp.dot(pooled / ms, w1_ref[...], preferred_element_type=F32)
        h = jnp.maximum(h + b1_ref[...], 0.0)
        out_ref[...] = jnp.dot(h, w2_ref[...],
                               preferred_element_type=F32) + b2_ref[...]

    return pl.pallas_call(
        k,
        out_shape=jax.ShapeDtypeStruct((B, NCLS), F32),
    )(pool_u, cs, cc, mask, se_pad, ce_pad,
      W1, b1.reshape(1, -1), W2, b2.reshape(1, -1))


def kernel(shapes, colors, clusters, mask, shape_emb, color_emb, cluster_emb,
           W1, b1, W2, b2):
    pool_u, cs, cc = _pool_sc(shapes, colors, clusters, cluster_emb)
    npad = _NBIN - shape_emb.shape[0]
    se_pad = jnp.pad(shape_emb.at[0].set(0.0), ((0, npad), (0, 0)))
    ce_pad = jnp.pad(color_emb.at[0].set(0.0), ((0, npad), (0, 0)))
    return _mlp_tc(pool_u, cs, cc, mask, se_pad, ce_pad, W1, b1, W2, b2)


# speedup vs baseline: 37.7745x; 1.2114x over previous
import functools

import jax
import jax.numpy as jnp
from jax import lax
from jax.experimental import pallas as pl
from jax.experimental.pallas import tpu as pltpu
from jax.experimental.pallas import tpu_sc as plsc

F32 = jnp.float32
_EMB = 64
_LANES = 16
_LA, _LB = 104, 96
_CH = 16
_NBIN = 1024


def _pool_sc(shapes, colors, clusters, cluster_emb):
    B, L = shapes.shape
    assert L == _LA + _LB
    info = plsc.get_sparse_core_info()
    NC, NS = info.num_cores, info.num_subcores
    NW = NC * NS
    RPW = B // NW
    NCHUNK = RPW // _CH

    mesh = plsc.VectorSubcoreMesh(core_axis_name="c", subcore_axis_name="s")

    @functools.partial(
        pl.kernel,
        out_type=(jax.ShapeDtypeStruct((B, _EMB), F32),
                  jax.ShapeDtypeStruct((B // 8, 8, 8, 128), F32),
                  jax.ShapeDtypeStruct((B // 8, 8, 8, 128), F32)),
        mesh=mesh,
        compiler_params=pltpu.CompilerParams(use_tc_tiling_on_sc=False,
                                             needs_layout_passes=False),
        scratch_types=[
            pltpu.VMEM((2, 3, _CH, _LA), jnp.int32),
            pltpu.VMEM((2, 3, _CH, _LB), jnp.int32),
            pltpu.VMEM((2, L, _EMB), F32),
            pltpu.VMEM((RPW, _EMB), F32),
            pltpu.VMEM((_EMB,), F32),
            pltpu.VMEM((2, 2, 8, 128), F32),
            pltpu.SemaphoreType.DMA((2,)),
            pltpu.SemaphoreType.DMA((2,)),
            pltpu.SemaphoreType.DMA((2,)),
        ],
    )
    def k(shapes_h, colors_h, clusters_h, ue_h, pool_h, cs_h, cc_h,
          idxA, idxB, buf, pooled, t0, cnts, isem, gsem, csem):
        wid = lax.axis_index("s") * NC + lax.axis_index("c")
        base = wid * RPW
        idx_hs = (shapes_h, colors_h, clusters_h)
        cnt_hs = (cs_h, cc_h)

        pltpu.sync_copy(ue_h.at[0], t0)

        lane = lax.iota(jnp.int32, _LANES)
        zeros = jnp.zeros((_LANES,), F32)
        ones = zeros + 1.0
        tailm = lane >= 8
        tailf = jnp.where(tailm, ones, zeros)

        def stage_chunk(c, slot):
            row0 = pl.multiple_of(base + c * _CH, 8)
            for t in range(3):
                pltpu.async_copy(
                    idx_hs[t].at[pl.ds(row0, _CH), pl.ds(0, _LA)],
                    idxA.at[slot, t], isem.at[slot])
                pltpu.async_copy(
                    idx_hs[t].at[pl.ds(row0, _CH), pl.ds(_LA, _LB)],
                    idxB.at[slot, t], isem.at[slot])

        def wait_chunk(slot):
            for t in range(3):
                pltpu.make_async_copy(
                    idx_hs[t].at[pl.ds(0, _CH), pl.ds(0, _LA)],
                    idxA.at[slot, t], isem.at[slot]).wait()
                pltpu.make_async_copy(
                    idx_hs[t].at[pl.ds(0, _CH), pl.ds(_LA, _LB)],
                    idxB.at[slot, t], isem.at[slot]).wait()

        def launch_gathers(r, gslot, cslot):
            rr = r - (r // _CH) * _CH
            pltpu.async_copy(ue_h.at[idxA.at[cslot, 2, rr]],
                             buf.at[gslot, pl.ds(0, _LA)], gsem.at[gslot])
            pltpu.async_copy(ue_h.at[idxB.at[cslot, 2, rr]],
                             buf.at[gslot, pl.ds(_LA, _LB)], gsem.at[gslot])

        def wait_gathers(gslot, cslot):
            pltpu.make_async_copy(ue_h.at[idxA.at[cslot, 2, 0]],
                                  buf.at[gslot, pl.ds(0, _LA)],
                                  gsem.at[gslot]).wait()
            pltpu.make_async_copy(ue_h.at[idxB.at[cslot, 2, 0]],
                                  buf.at[gslot, pl.ds(_LA, _LB)],
                                  gsem.at[gslot]).wait()

        stage_chunk(0, 0)
        stage_chunk(1, 1)
        wait_chunk(0)
        launch_gathers(0, 0, 0)

        def row_body(r, carry):
            gslot = lax.rem(r, 2)
            chunk = r // _CH
            cslot = lax.rem(chunk, 2)
            rr = r - chunk * _CH
            row = base + r

            nxt = r + 1

            @pl.when(nxt < RPW)
            def _():
                nchunk = nxt // _CH
                ncslot = lax.rem(nchunk, 2)

                @pl.when(nxt == nchunk * _CH)
                def _():
                    wait_chunk(ncslot)

                launch_gathers(nxt, lax.rem(nxt, 2), ncslot)

            wait_gathers(gslot, cslot)

            def tok_body(l, accs):
                a = list(accs)
                for u in range(4):
                    for c in range(4):
                        a[c] = a[c] + buf[gslot, 4 * l + u,
                                          pl.ds(c * _LANES, _LANES)]
                return tuple(a)

            accs = lax.fori_loop(
                0, L // 4, tok_body, tuple(zeros for _ in range(4)))

            cnt = zeros
            for kk in range(6):
                iv = idxA[cslot, 2, rr, pl.ds(kk * _LANES, _LANES)]
                cnt = cnt + jnp.where(iv == 0, ones, zeros)
            iv = idxA[cslot, 2, rr, pl.ds(_LA - _LANES, _LANES)]
            cnt = cnt + jnp.where(iv == 0, tailf, zeros)
            for kk in range(6):
                iv = idxB[cslot, 2, rr, pl.ds(kk * _LANES, _LANES)]
                cnt = cnt + jnp.where(iv == 0, ones, zeros)
            n0 = lax.broadcast_in_dim(jnp.sum(cnt), (_LANES,), ())
            for c in range(4):
                pooled[r, pl.ds(c * _LANES, _LANES)] = (
                    accs[c] - n0 * t0[pl.ds(c * _LANES, _LANES)])

            row8 = row // 8
            rowm = row - row8 * 8

            @pl.when(r >= 2)
            def _():
                for t in range(2):
                    pltpu.make_async_copy(
                        cnts.at[gslot, t],
                        cnt_hs[t].at[0, pl.ds(0, 8), 0],
                        csem.at[gslot]).wait()

            for t in range(2):
                cref = cnts.at[gslot, t]
                for i in range(8):
                    for kk in range(8):
                        cref[i, pl.ds(kk * _LANES, _LANES)] = zeros
                for kk in range(6):
                    iv = idxA[cslot, t, rr, pl.ds(kk * _LANES, _LANES)]
                    plsc.addupdate_scatter(
                        cref, [iv >> 7, iv & 127], ones)
                iv = idxA[cslot, t, rr, pl.ds(_LA - _LANES, _LANES)]
                plsc.addupdate_scatter(
                    cref, [iv >> 7, iv & 127], ones, mask=tailm)
                for kk in range(6):
                    iv = idxB[cslot, t, rr, pl.ds(kk * _LANES, _LANES)]
                    plsc.addupdate_scatter(
                        cref, [iv >> 7, iv & 127], ones)
                pltpu.async_copy(cnts.at[gslot, t],
                                 cnt_hs[t].at[row8, pl.ds(0, 8), rowm],
                                 csem.at[gslot])

            @pl.when((rr == _CH - 1) & (chunk + 2 < NCHUNK))
            def _():
                stage_chunk(chunk + 2, cslot)

            return carry

        lax.fori_loop(0, RPW, row_body, 0)

        for s in range(2):
            for t in range(2):
                pltpu.make_async_copy(
                    cnts.at[s, t], cnt_hs[t].at[0, pl.ds(0, 8), 0],
                    csem.at[s]).wait()

        pltpu.sync_copy(pooled, pool_h.at[pl.ds(pl.multiple_of(base, 8), RPW)])

    return k(shapes, colors, clusters, cluster_emb)


def _mlp_tc(pool_u, cs4, cc4, mask, se_pad, ce_pad, W1, b1, W2, b2):
    B = pool_u.shape[0]
    NCLS = W2.shape[1]

    def k(pu_ref, cs_ref, cc_ref, mask_ref, se_ref, ce_ref,
          w1_ref, b1_ref, w2_ref, b2_ref, out_ref):
        ps = jnp.zeros((B, _EMB), F32)
        pc = jnp.zeros((B, _EMB), F32)
        cs = cs_ref[...]
        cc = cc_ref[...]
        se = se_ref[...]
        ce = ce_ref[...]
        for j in range(_NBIN // 128):
            tabs = se[j * 128:(j + 1) * 128, :]
            tabc = ce[j * 128:(j + 1) * 128, :]
            ps = ps + jnp.dot(cs[:, j].reshape(B, 128), tabs,
                              preferred_element_type=F32)
            pc = pc + jnp.dot(cc[:, j].reshape(B, 128), tabc,
                              preferred_element_type=F32)
        pooled = jnp.concatenate([ps, pc, pu_ref[...]], axis=1)
        ms = jnp.sum(mask_ref[...], axis=1, keepdims=True)
        h = jnp.dot(pooled / ms, w1_ref[...], preferred_element_type=F32)
        h = jnp.maximum(h + b1_ref[...], 0.0)
        out_ref[...] = jnp.dot(h, w2_ref[...],
                               preferred_element_type=F32) + b2_ref[...]

    return pl.pallas_call(
        k,
        out_shape=jax.ShapeDtypeStruct((B, NCLS), F32),
    )(pool_u, cs4, cc4, mask, se_pad, ce_pad,
      W1, b1.reshape(1, -1), W2, b2.reshape(1, -1))


def kernel(shapes, colors, clusters, mask, shape_emb, color_emb, cluster_emb,
           W1, b1, W2, b2):
    pool_u, cs4, cc4 = _pool_sc(shapes, colors, clusters, cluster_emb)
    npad = _NBIN - shape_emb.shape[0]
    se_pad = jnp.pad(shape_emb.at[0].set(0.0), ((0, npad), (0, 0)))
    ce_pad = jnp.pad(color_emb.at[0].set(0.0), ((0, npad), (0, 0)))
    return _mlp_tc(pool_u, cs4, cc4, mask, se_pad, ce_pad, W1, b1, W2, b2)
